# baseline (speedup 1.0000x reference)
import functools

import jax
import jax.numpy as jnp
from jax import lax
from jax.experimental import pallas as pl
from jax.experimental.pallas import tpu as pltpu
from jax.experimental.pallas import tpu_sc as plsc


def _vq_block_kernel(x_ref, w_ref, enc_ref, idx_ref, loss_ref,
                     w2_ref, iota_ref, acc_ref, *, n_total):
    step = pl.program_id(0)
    nsteps = pl.num_programs(0)
    bm, k = enc_ref.shape

    @pl.when(step == 0)
    def _init():
        w = w_ref[...]
        ones_row = jnp.ones((1, w.shape[1]), jnp.float32)
        w2_ref[...] = jax.lax.dot_general(
            ones_row, w * w,
            dimension_numbers=(((1,), (1,)), ((), ())),
            preferred_element_type=jnp.float32)
        iota_ref[...] = jax.lax.broadcasted_iota(
            jnp.int32, (1, k), 1).astype(jnp.float32)
        acc_ref[0, 0] = 0.0

    x = x_ref[...]
    t1 = jnp.sum(x * x, axis=1, keepdims=True)
    t3 = jax.lax.dot_general(
        x, w_ref[...],
        dimension_numbers=(((1,), (1,)), ((), ())),
        preferred_element_type=jnp.float32)
    w2 = w2_ref[...]
    iota = iota_ref[...]
    lanes = 128
    mval = None
    midx = None
    iota_l = iota[:, :lanes]
    for s in range(k // lanes):
        dsl = (t1 + w2[:, s * lanes:(s + 1) * lanes]
               ) - 2.0 * t3[:, s * lanes:(s + 1) * lanes]
        if mval is None:
            mval, midx = dsl, jnp.zeros((bm, lanes), jnp.float32) + iota_l
        else:
            upd = dsl < mval
            mval = jnp.where(upd, dsl, mval)
            midx = jnp.where(upd, iota_l + float(s * lanes), midx)
    dmin = jnp.min(mval, axis=1, keepdims=True)
    masked = jnp.where(mval == dmin, midx, float(k))
    idx = jnp.min(masked, axis=1, keepdims=True)
    enc = jnp.where(iota == idx, 1.0, 0.0)
    enc_ref[...] = enc
    idx_ref[...] = idx.astype(jnp.int32)
    acc_ref[0, 0] += jnp.sum(dmin)

    @pl.when(step == nsteps - 1)
    def _fin():
        loss_ref[0, 0] = 1.25 * (acc_ref[0, 0] / n_total)


def _sc_gather_rows(table_hbm, idx_hbm, out_hbm, idx_v, rows_v, sem,
                    *, n_workers, rows_per_worker, chunk):
    wid = lax.axis_index("s") * 2 + lax.axis_index("c")
    base = wid * rows_per_worker
    for c in range(rows_per_worker // chunk):
        off = base + c * chunk
        pltpu.sync_copy(idx_hbm.at[pl.ds(off, chunk)], idx_v)
        pltpu.async_copy(table_hbm.at[idx_v], rows_v, sem).wait()
        pltpu.sync_copy(rows_v, out_hbm.at[pl.ds(off, chunk)])


def kernel(inputs, W, n, fine_tuning):
    input_shape = inputs.shape
    d = input_shape[-1]
    flat = inputs.reshape(-1, d)
    m = flat.shape[0]
    k = W.shape[0]
    bm = 512
    grid = (m // bm,)

    enc, idx, loss = pl.pallas_call(
        functools.partial(_vq_block_kernel, n_total=float(flat.size)),
        grid=grid,
        in_specs=[
            pl.BlockSpec((bm, d), lambda i: (i, 0)),
            pl.BlockSpec((k, d), lambda i: (0, 0)),
        ],
        out_specs=[
            pl.BlockSpec((bm, k), lambda i: (i, 0)),
            pl.BlockSpec((bm, 1), lambda i: (i, 0)),
            pl.BlockSpec(memory_space=pltpu.SMEM),
        ],
        out_shape=[
            jax.ShapeDtypeStruct((m, k), jnp.float32),
            jax.ShapeDtypeStruct((m, 1), jnp.int32),
            jax.ShapeDtypeStruct((1, 1), jnp.float32),
        ],
        scratch_shapes=[
            pltpu.VMEM((1, k), jnp.float32),
            pltpu.VMEM((1, k), jnp.float32),
            pltpu.SMEM((1, 1), jnp.float32),
        ],
        compiler_params=pltpu.CompilerParams(
            dimension_semantics=("arbitrary",),
        ),
    )(flat, W)

    info = plsc.get_sparse_core_info()
    n_workers = info.num_cores * info.num_subcores
    rows_per_worker = m // n_workers
    chunk = min(256, rows_per_worker)

    gather = functools.partial(
        pl.kernel,
        mesh=plsc.VectorSubcoreMesh(core_axis_name="c", subcore_axis_name="s"),
        out_type=jax.ShapeDtypeStruct((m, d), jnp.float32),
        scratch_types=[
            pltpu.VMEM((chunk,), jnp.int32),
            pltpu.VMEM((chunk, d), jnp.float32),
            pltpu.SemaphoreType.DMA,
        ],
    )(functools.partial(
        _sc_gather_rows, n_workers=n_workers,
        rows_per_worker=rows_per_worker, chunk=chunk))

    q = gather(W, idx.reshape(m))

    return (loss[0, 0], q.reshape(input_shape), enc)


# speedup vs baseline: 1.0213x; 1.0213x over previous
import functools

import jax
import jax.numpy as jnp
from jax import lax
from jax.experimental import pallas as pl
from jax.experimental.pallas import tpu as pltpu
from jax.experimental.pallas import tpu_sc as plsc


def _vq_block_kernel(x_ref, w_ref, enc_ref, idx_ref, loss_ref,
                     w2_ref, iota_ref, acc_ref, *, n_total):
    step = pl.program_id(0)
    nsteps = pl.num_programs(0)
    bm, k = enc_ref.shape

    @pl.when(step == 0)
    def _init():
        w = w_ref[...]
        ones_row = jnp.ones((1, w.shape[1]), jnp.float32)
        w2_ref[...] = jax.lax.dot_general(
            ones_row, w * w,
            dimension_numbers=(((1,), (1,)), ((), ())),
            preferred_element_type=jnp.float32)
        iota_ref[...] = jax.lax.broadcasted_iota(
            jnp.int32, (1, k), 1).astype(jnp.float32)
        acc_ref[0, 0] = 0.0

    x = x_ref[...]
    t1 = jnp.sum(x * x, axis=1, keepdims=True)
    t3 = jax.lax.dot_general(
        x, w_ref[...],
        dimension_numbers=(((1,), (1,)), ((), ())),
        preferred_element_type=jnp.float32)
    d = (t1 + w2_ref[...]) - 2.0 * t3
    dmin = jnp.min(d, axis=1, keepdims=True)
    iota = iota_ref[...]
    masked = jnp.where(d == dmin, iota, float(k))
    idx = jnp.min(masked, axis=1, keepdims=True)
    enc = jnp.where(iota == idx, 1.0, 0.0)
    enc_ref[...] = enc
    idx_ref[...] = idx.astype(jnp.int32)
    acc_ref[0, 0] += jnp.sum(dmin)

    @pl.when(step == nsteps - 1)
    def _fin():
        loss_ref[0, 0] = 1.25 * (acc_ref[0, 0] / n_total)


def _sc_gather_rows(table_hbm, idx_hbm, out_hbm, idx_v0, rows_v0, sem0,
                    idx_v1, rows_v1, sem1,
                    *, n_workers, rows_per_worker, chunk):
    wid = lax.axis_index("s") * 2 + lax.axis_index("c")
    base = wid * rows_per_worker
    nch = rows_per_worker // chunk
    bufs = ((idx_v0, rows_v0, sem0), (idx_v1, rows_v1, sem1))

    def fire(c):
        ib, rb, sm = bufs[c % 2]
        pltpu.sync_copy(idx_hbm.at[pl.ds(base + c * chunk, chunk)], ib)
        return pltpu.async_copy(table_hbm.at[ib], rb, sm)

    pending = fire(0)
    for c in range(nch):
        nxt = fire(c + 1) if c + 1 < nch else None
        pending.wait()
        pltpu.sync_copy(bufs[c % 2][1], out_hbm.at[pl.ds(base + c * chunk, chunk)])
        pending = nxt


def kernel(inputs, W, n, fine_tuning):
    input_shape = inputs.shape
    d = input_shape[-1]
    flat = inputs.reshape(-1, d)
    m = flat.shape[0]
    k = W.shape[0]
    bm = 512
    grid = (m // bm,)

    enc, idx, loss = pl.pallas_call(
        functools.partial(_vq_block_kernel, n_total=float(flat.size)),
        grid=grid,
        in_specs=[
            pl.BlockSpec((bm, d), lambda i: (i, 0)),
            pl.BlockSpec((k, d), lambda i: (0, 0)),
        ],
        out_specs=[
            pl.BlockSpec((bm, k), lambda i: (i, 0)),
            pl.BlockSpec((bm, 1), lambda i: (i, 0)),
            pl.BlockSpec(memory_space=pltpu.SMEM),
        ],
        out_shape=[
            jax.ShapeDtypeStruct((m, k), jnp.float32),
            jax.ShapeDtypeStruct((m, 1), jnp.int32),
            jax.ShapeDtypeStruct((1, 1), jnp.float32),
        ],
        scratch_shapes=[
            pltpu.VMEM((1, k), jnp.float32),
            pltpu.VMEM((1, k), jnp.float32),
            pltpu.SMEM((1, 1), jnp.float32),
        ],
        compiler_params=pltpu.CompilerParams(
            dimension_semantics=("arbitrary",),
        ),
    )(flat, W)

    info = plsc.get_sparse_core_info()
    n_workers = info.num_cores * info.num_subcores
    rows_per_worker = m // n_workers
    chunk = min(128, rows_per_worker)

    gather = functools.partial(
        pl.kernel,
        mesh=plsc.VectorSubcoreMesh(core_axis_name="c", subcore_axis_name="s"),
        out_type=jax.ShapeDtypeStruct((m, d), jnp.float32),
        scratch_types=[
            pltpu.VMEM((chunk,), jnp.int32),
            pltpu.VMEM((chunk, d), jnp.float32),
            pltpu.SemaphoreType.DMA,
            pltpu.VMEM((chunk,), jnp.int32),
            pltpu.VMEM((chunk, d), jnp.float32),
            pltpu.SemaphoreType.DMA,
        ],
    )(functools.partial(
        _sc_gather_rows, n_workers=n_workers,
        rows_per_worker=rows_per_worker, chunk=chunk))

    q = gather(W, idx.reshape(m))

    return (loss[0, 0], q.reshape(input_shape), enc)
